# baseline (speedup 1.0000x reference)
import functools

import jax
import jax.numpy as jnp
from jax import lax
from jax.experimental import pallas as pl
from jax.experimental.pallas import tpu as pltpu
from jax.experimental.pallas import tpu_sc as plsc

E = 320000
D = 128
V = 20

NW = 32
PER_W = E // NW
GROUP = 80
NBUF = 5
BATCH = GROUP * NBUF
NOUTER = PER_W // BATCH


def _combos_body(e0_ref, e1_ref, e2_ref, out_ref):
    t01 = e0_ref[...][:, None, None, :] + e1_ref[...][None, :, None, :]
    blk = t01 + e2_ref[...][None, None, :, :]
    out_ref[...] = blk.reshape(V * V * V, D)


def _combos(emb0, emb1, emb2):
    return pl.pallas_call(
        _combos_body,
        out_shape=jax.ShapeDtypeStruct((V * V * V, D), jnp.float32),
    )(emb0, emb1, emb2)


@functools.partial(
    pl.kernel,
    mesh=plsc.VectorSubcoreMesh(core_axis_name="c", subcore_axis_name="s"),
    out_type=jax.ShapeDtypeStruct((E, D), jnp.float32),
    scratch_types=(
        [pltpu.VMEM((3 * PER_W,), jnp.int32)]
        + [pltpu.VMEM((PER_W,), jnp.int32)]
        + [pltpu.VMEM((GROUP, D), jnp.float32)] * NBUF
        + [pltpu.SemaphoreType.DMA] * (1 + 2 * NBUF)
    ),
)
def _sc_gather(attr_hbm, combos_hbm, out_hbm, ai_v, cidx_v, *bufs_and_sems):
    rows = bufs_and_sems[:NBUF]
    isem = bufs_and_sems[NBUF]
    gsem = bufs_and_sems[NBUF + 1:2 * NBUF + 1]
    ssem = bufs_and_sems[2 * NBUF + 1:]
    wid = lax.axis_index("s") * 2 + lax.axis_index("c")
    base = wid * PER_W

    pltpu.async_copy(attr_hbm.at[pl.ds(3 * base, 3 * PER_W)], ai_v,
                     isem).wait()

    lane = lax.iota(jnp.int32, 16)

    dnums = lax.GatherDimensionNumbers(
        offset_dims=(), collapsed_slice_dims=(0,), start_index_map=(0,))

    def _dg(v, idx):
        return lax.gather(
            v, idx[:, None], dimension_numbers=dnums, slice_sizes=(1,),
            mode=lax.GatherScatterMode.PROMISE_IN_BOUNDS)

    def _pick(v0, v1, v2, p):
        g0 = _dg(v0, jnp.minimum(p, 15))
        g1 = _dg(v1, jnp.clip(p - 16, 0, 15))
        g2 = _dg(v2, jnp.clip(p - 32, 0, 15))
        return jnp.where(p < 16, g0, jnp.where(p < 32, g1, g2))

    def fuse_batch(o):
        def fuse_body(j, carry):
            woff = pl.multiple_of(j * 48, 48)
            v0 = ai_v[pl.ds(woff, 16)]
            v1 = ai_v[pl.ds(woff + 16, 16)]
            v2 = ai_v[pl.ds(woff + 32, 16)]
            p = lane * 3
            c = (_pick(v0, v1, v2, p) * 400
                 + _pick(v0, v1, v2, p + 1) * 20
                 + _pick(v0, v1, v2, p + 2))
            cidx_v[pl.ds(pl.multiple_of(j * 16, 16), 16)] = c
            return carry
        lax.fori_loop(o * (BATCH // 16), (o + 1) * (BATCH // 16),
                      fuse_body, 0)

    fuse_batch(0)

    def outer_body(o, carry):
        goff = pl.multiple_of(o * BATCH, BATCH)
        gcps = []
        for b in range(NBUF):
            @pl.when(o > 0)
            def _(b=b):
                pltpu.make_async_copy(
                    rows[b], out_hbm.at[pl.ds(0, GROUP)], ssem[b]).wait()
            cidx_sl = cidx_v.at[pl.ds(goff + b * GROUP, GROUP)]
            gcps.append(pltpu.async_copy(
                combos_hbm.at[cidx_sl], rows[b], gsem[b]))
        @pl.when(o + 1 < NOUTER)
        def _():
            fuse_batch(o + 1)
        for b in range(NBUF):
            gcps[b].wait()
            pltpu.async_copy(
                rows[b], out_hbm.at[pl.ds(base + goff + b * GROUP, GROUP)],
                ssem[b])
        return carry

    lax.fori_loop(0, NOUTER, outer_body, 0)
    for b in range(NBUF):
        pltpu.make_async_copy(
            rows[b], out_hbm.at[pl.ds(0, GROUP)], ssem[b]).wait()


def kernel(edge_attr, emb0, emb1, emb2):
    combos = _combos(emb0, emb1, emb2)
    return _sc_gather(edge_attr.reshape(3 * E), combos)


# speedup vs baseline: 2.2731x; 2.2731x over previous
import functools

import jax
import jax.numpy as jnp
from jax import lax
from jax.experimental import pallas as pl
from jax.experimental.pallas import tpu as pltpu
from jax.experimental.pallas import tpu_sc as plsc

E = 320000
D = 128
V = 20

NW = 32
PER_W = E // NW
GROUP = 80
NBUF = 5
BATCH = GROUP * NBUF
NOUTER = PER_W // BATCH


def _combos_body(e0_ref, e1_ref, e2_ref, out_ref):
    t01 = e0_ref[...][:, None, None, :] + e1_ref[...][None, :, None, :]
    blk = t01 + e2_ref[...][None, None, :, :]
    out_ref[...] = blk.reshape(V * V * V, D)


def _combos(emb0, emb1, emb2):
    return pl.pallas_call(
        _combos_body,
        out_shape=jax.ShapeDtypeStruct((V * V * V, D), jnp.float32),
    )(emb0, emb1, emb2)


@functools.partial(
    pl.kernel,
    mesh=plsc.VectorSubcoreMesh(core_axis_name="c", subcore_axis_name="s"),
    out_type=jax.ShapeDtypeStruct((E, D), jnp.float32),
    scratch_types=(
        [pltpu.VMEM((PER_W,), jnp.int32)] * 4
        + [pltpu.VMEM((GROUP, D), jnp.float32)] * NBUF
        + [pltpu.SemaphoreType.DMA] * (1 + 2 * NBUF)
    ),
)
def _sc_gather(attr_hbm, combos_hbm, out_hbm,
               a0_v, a1_v, a2_v, cidx_v, *bufs_and_sems):
    rows = bufs_and_sems[:NBUF]
    isem = bufs_and_sems[NBUF]
    gsem = bufs_and_sems[NBUF + 1:2 * NBUF + 1]
    ssem = bufs_and_sems[2 * NBUF + 1:]
    wid = lax.axis_index("s") * 2 + lax.axis_index("c")
    base = wid * PER_W

    cps = [pltpu.async_copy(attr_hbm.at[pl.ds(k * E + base, PER_W)],
                            v, isem)
           for k, v in ((0, a0_v), (1, a1_v), (2, a2_v))]
    for cp in cps:
        cp.wait()

    def fuse_batch(o):
        def fuse_body(j, carry):
            sl = pl.ds(pl.multiple_of(j * 16, 16), 16)
            cidx_v[sl] = a0_v[sl] * 400 + a1_v[sl] * 20 + a2_v[sl]
            return carry
        lax.fori_loop(o * (BATCH // 16), (o + 1) * (BATCH // 16),
                      fuse_body, 0)

    fuse_batch(0)

    def outer_body(o, carry):
        goff = pl.multiple_of(o * BATCH, BATCH)
        gcps = []
        for b in range(NBUF):
            @pl.when(o > 0)
            def _(b=b):
                pltpu.make_async_copy(
                    rows[b], out_hbm.at[pl.ds(0, GROUP)], ssem[b]).wait()
            cidx_sl = cidx_v.at[pl.ds(goff + b * GROUP, GROUP)]
            gcps.append(pltpu.async_copy(
                combos_hbm.at[cidx_sl], rows[b], gsem[b]))
        @pl.when(o + 1 < NOUTER)
        def _():
            fuse_batch(o + 1)
        for b in range(NBUF):
            gcps[b].wait()
            pltpu.async_copy(
                rows[b], out_hbm.at[pl.ds(base + goff + b * GROUP, GROUP)],
                ssem[b])
        return carry

    lax.fori_loop(0, NOUTER, outer_body, 0)
    for b in range(NBUF):
        pltpu.make_async_copy(
            rows[b], out_hbm.at[pl.ds(0, GROUP)], ssem[b]).wait()


def kernel(edge_attr, emb0, emb1, emb2):
    combos = _combos(emb0, emb1, emb2)
    return _sc_gather(edge_attr.T.reshape(3 * E), combos)
